# baseline (speedup 1.0000x reference)
import functools

import jax
import jax.numpy as jnp
from jax import lax
from jax.experimental import pallas as pl
from jax.experimental.pallas import tpu as pltpu
from jax.experimental.pallas import tpu_sc as plsc

F = 26
D = 16
FD = F * D
P = (F * (F - 1)) // 2
OUT_W = FD + P
B = 16384
NW = 32
ROWS_PER_W = B // NW
CHUNK = 64
GROUPS = CHUNK // 16
NCHUNK = ROWS_PER_W // CHUNK


def _sc_body(x_hbm, out_hbm, in_buf, out_buf):
    info = plsc.get_sparse_core_info()
    nc = info.num_cores
    wid = lax.axis_index("s") * nc + lax.axis_index("c")
    lane = lax.iota(jnp.int32, 16)

    def chunk_body(c, carry):
        rowbase = wid * ROWS_PER_W + c * CHUNK
        pltpu.sync_copy(x_hbm.at[pl.ds(rowbase * FD, CHUNK * FD)], in_buf)

        def group_body(g, carry2):
            rowvec = lane + g * 16
            inbase = rowvec * FD
            outbase = rowvec * OUT_W

            def copy_body(w, cc):
                for d in range(16):
                    off = w * 16 + d
                    v = plsc.load_gather(in_buf, [inbase + off])
                    plsc.store_scatter(out_buf, [outbase + off], v)
                return cc

            lax.fori_loop(0, F, copy_body, 0)

            def i_body(i, ci):
                vi = [plsc.load_gather(in_buf, [inbase + i * 16 + d])
                      for d in range(16)]
                pbase = FD + i * 24 - (i * (i - 1)) // 2 - 1

                def j_body(j, cj):
                    acc = vi[0] * plsc.load_gather(in_buf, [inbase + j * 16])
                    for d in range(1, 16):
                        acc = acc + vi[d] * plsc.load_gather(
                            in_buf, [inbase + j * 16 + d])
                    plsc.store_scatter(out_buf, [outbase + pbase + j], acc)
                    return cj

                lax.fori_loop(i + 1, F, j_body, 0)
                return ci

            lax.fori_loop(0, F - 1, i_body, 0)
            return carry2

        lax.fori_loop(0, GROUPS, group_body, 0)
        pltpu.sync_copy(out_buf, out_hbm.at[pl.ds(rowbase * OUT_W, CHUNK * OUT_W)])
        return carry

    lax.fori_loop(0, NCHUNK, chunk_body, 0)


@jax.jit
def _run(xf):
    mesh = plsc.VectorSubcoreMesh(core_axis_name="c", subcore_axis_name="s")
    return pl.kernel(
        _sc_body,
        mesh=mesh,
        out_type=jax.ShapeDtypeStruct((B * OUT_W,), jnp.float32),
        scratch_types=[
            pltpu.VMEM((CHUNK * FD,), jnp.float32),
            pltpu.VMEM((CHUNK * OUT_W,), jnp.float32),
        ],
        compiler_params=pltpu.CompilerParams(needs_layout_passes=False),
    )(xf)


def kernel(x):
    xf = x.reshape(-1)
    out = _run(xf)
    return out.reshape(B, OUT_W)
